# Initial kernel scaffold; baseline (speedup 1.0000x reference)
import jax
import jax.numpy as jnp
from jax.experimental import pallas as pl


def kernel(in_, edge_index, labels, weights, W_feast, U, c, b_feast, W1, b1, W2, b2):
    raise NotImplementedError("write your pallas kernel here")


# speedup vs baseline: 14.4013x; 14.4013x over previous
import functools

import jax
import jax.numpy as jnp
from jax import lax
from jax.experimental import pallas as pl
from jax.experimental.pallas import tpu as pltpu
from jax.experimental.pallas import tpu_sc as plsc

N_NODES = 10000
N_EDGES = 320000
D_IN = 128
D_HID = 16

NC = 2
NS = 16
NW = NC * NS
LANES = 16

BE = 128
N_PAD = 10240
STRIPE = N_PAD // NS
TRASH_ROW = N_NODES
BLOCKS_PER_W = 79
E_PAD = NW * BE * BLOCKS_PER_W
EW = E_PAD // NW


def _matmul_body(x_ref, w_ref, y_ref):
    y_ref[...] = jnp.dot(x_ref[...], w_ref[...],
                         preferred_element_type=jnp.float32)


def _edge_body(y_hbm, src_hbm, dst_hbm, agg_out, deg_out,
               src_buf, dst_buf, dstp_buf, rows_buf, ones_buf, stage_buf,
               agg_sh, deg_sh):
    cid = lax.axis_index("c")
    sid = lax.axis_index("s")
    wid = cid * NS + sid

    zeros16 = jnp.zeros((LANES,), jnp.float32)
    ones16 = jnp.ones((LANES,), jnp.float32)

    def _init_row(i, _):
        ones_buf[i, :] = ones16
        return 0
    lax.fori_loop(0, BE, _init_row, 0)

    def _zero_row(i, _):
        stage_buf[i, :] = zeros16
        return 0
    lax.fori_loop(0, STRIPE, _zero_row, 0)

    row0 = sid * STRIPE

    pltpu.sync_copy(stage_buf, agg_sh.at[pl.ds(row0, STRIPE)])
    pltpu.sync_copy(stage_buf, deg_sh.at[pl.ds(row0, STRIPE)])
    plsc.subcore_barrier()

    def _block(j, _):
        base = wid * EW + j * BE
        pltpu.sync_copy(src_hbm.at[pl.ds(base, BE)], src_buf)
        pltpu.sync_copy(dst_hbm.at[pl.ds(base, BE)], dst_buf)
        for i in range(BE // LANES):
            s = src_buf[pl.ds(i * LANES, LANES)]
            d = dst_buf[pl.ds(i * LANES, LANES)]
            dstp_buf[pl.ds(i * LANES, LANES)] = jnp.where(
                s == d, TRASH_ROW, d)
        pltpu.sync_copy(y_hbm.at[src_buf], rows_buf)
        pltpu.sync_copy(rows_buf, agg_sh.at[dstp_buf], add=True)
        pltpu.sync_copy(ones_buf, deg_sh.at[dstp_buf], add=True)
        return 0

    lax.fori_loop(0, BLOCKS_PER_W, _block, 0)
    plsc.subcore_barrier()

    pltpu.sync_copy(agg_sh.at[pl.ds(row0, STRIPE)], stage_buf)
    pltpu.sync_copy(stage_buf, agg_out.at[cid, pl.ds(row0, STRIPE)])
    pltpu.sync_copy(deg_sh.at[pl.ds(row0, STRIPE)], stage_buf)
    pltpu.sync_copy(stage_buf, deg_out.at[cid, pl.ds(row0, STRIPE)])


def _post_body(y_ref, agg_ref, deg_ref, labels_ref, weights_ref,
               bf_ref, w1_ref, b1_ref, w2_ref, b2_ref,
               p_ref, loss_ref):
    agg = agg_ref[0] + agg_ref[1] + y_ref[...]
    deg = deg_ref[0] + deg_ref[1] + 1.0
    h = agg / deg + bf_ref[...]
    h = jnp.maximum(h, 0.0)
    h = jnp.dot(h, w1_ref[...], preferred_element_type=jnp.float32)
    h = jnp.maximum(h + b1_ref[...], 0.0)
    z = jnp.dot(h, w2_ref[...], preferred_element_type=jnp.float32)
    z = z + b2_ref[...]
    p = jax.nn.sigmoid(z)
    log_p = jnp.clip(jnp.log(p), -100.0)
    log_1mp = jnp.clip(jnp.log(1.0 - p), -100.0)
    lab = labels_ref[...]
    per = weights_ref[...] * -(lab * log_p + (1.0 - lab) * log_1mp)
    rows = lax.broadcasted_iota(jnp.int32, (N_PAD, 1), 0)
    per = jnp.where(rows < N_NODES, per, 0.0)
    p_ref[...] = p
    loss_ref[...] = (jnp.sum(per) / N_NODES).reshape(1, 1)


def kernel(in_, edge_index, labels, weights, W_feast, U, c, b_feast,
           W1, b1, W2, b2):
    del U, c

    x_pad = jnp.zeros((N_PAD, D_IN), jnp.float32).at[:N_NODES].set(in_)
    y_pad = pl.pallas_call(
        _matmul_body,
        out_shape=jax.ShapeDtypeStruct((N_PAD, D_HID), jnp.float32),
    )(x_pad, W_feast)

    src = edge_index[0].astype(jnp.int32)
    dst = edge_index[1].astype(jnp.int32)
    padn = E_PAD - N_EDGES
    zpad = jnp.zeros((padn,), jnp.int32)
    src_p = jnp.concatenate([src, zpad])
    dst_p = jnp.concatenate([dst, zpad])

    mesh = plsc.VectorSubcoreMesh(core_axis_name="c", subcore_axis_name="s",
                                  num_cores=NC, num_subcores=NS)
    agg_parts, deg_parts = pl.kernel(
        _edge_body,
        out_type=[
            jax.ShapeDtypeStruct((NC, N_PAD, D_HID), jnp.float32),
            jax.ShapeDtypeStruct((NC, N_PAD, D_HID), jnp.float32),
        ],
        mesh=mesh,
        compiler_params=pltpu.CompilerParams(use_tc_tiling_on_sc=False),
        scratch_types=[
            pltpu.VMEM((BE,), jnp.int32),
            pltpu.VMEM((BE,), jnp.int32),
            pltpu.VMEM((BE,), jnp.int32),
            pltpu.VMEM((BE, D_HID), jnp.float32),
            pltpu.VMEM((BE, D_HID), jnp.float32),
            pltpu.VMEM((STRIPE, D_HID), jnp.float32),
            pltpu.VMEM_SHARED((N_PAD, D_HID), jnp.float32),
            pltpu.VMEM_SHARED((N_PAD, D_HID), jnp.float32),
        ],
    )(y_pad, src_p, dst_p)

    lab_pad = jnp.zeros((N_PAD, 1), jnp.float32).at[:N_NODES].set(labels)
    wt_pad = jnp.zeros((N_PAD, 1), jnp.float32).at[:N_NODES].set(weights)

    p_pad, loss_out = pl.pallas_call(
        _post_body,
        out_shape=[
            jax.ShapeDtypeStruct((N_PAD, 1), jnp.float32),
            jax.ShapeDtypeStruct((1, 1), jnp.float32),
        ],
    )(y_pad, agg_parts, deg_parts, lab_pad, wt_pad,
      b_feast.reshape(1, D_HID), W1, b1.reshape(1, 8), W2, b2.reshape(1, 1))

    return loss_out[0, 0], p_pad[:N_NODES]


# speedup vs baseline: 22.2535x; 1.5452x over previous
import jax
import jax.numpy as jnp
from jax import lax
from jax.experimental import pallas as pl
from jax.experimental.pallas import tpu as pltpu
from jax.experimental.pallas import tpu_sc as plsc

N_NODES = 10000
N_EDGES = 320000
D_IN = 128
D_HID = 16

NC = 2
NS = 16
NW = NC * NS
LANES = 16

BE = 128
DW = 8
KBUF = 8
NBLK = 80
N_PAD = 10240
STRIPE = N_PAD // NS
E_PAD = NW * NBLK * BE
EW = NBLK * BE


def _matmul_body(x_ref, w_ref, y_ref):
    y_ref[...] = jnp.dot(x_ref[...], w_ref[...],
                         preferred_element_type=jnp.float32)


def _edge_body(y_hbm, src_hbm, dst_hbm, ones_hbm, z16_hbm, z8_hbm,
               agg_out, deg_out,
               src_all, dst_all, ones_buf, stage_buf, deg_stage,
               rows0, rows1, rows2, rows3, rows4, rows5, rows6, rows7,
               sem0, sem1, sem2, sem3, sem4, sem5, sem6, sem7,
               agg_sh, deg_sh):
    cid = lax.axis_index("c")
    sid = lax.axis_index("s")
    wid = cid * NS + sid
    trash = N_NODES + wid
    row0 = sid * STRIPE
    rows = (rows0, rows1, rows2, rows3, rows4, rows5, rows6, rows7)
    sems = (sem0, sem1, sem2, sem3, sem4, sem5, sem6, sem7)

    pltpu.sync_copy(src_hbm.at[wid], src_all)
    pltpu.sync_copy(dst_hbm.at[wid], dst_all)

    pltpu.sync_copy(ones_hbm, ones_buf)
    pltpu.sync_copy(z16_hbm, stage_buf)
    pltpu.sync_copy(z8_hbm, deg_stage)
    pltpu.sync_copy(stage_buf, agg_sh.at[pl.ds(row0, STRIPE)])
    pltpu.sync_copy(deg_stage, deg_sh.at[pl.ds(row0, STRIPE)])

    def _remap(j, _):
        for i in range(BE // LANES):
            s = src_all[j, pl.ds(i * LANES, LANES)]
            d = dst_all[j, pl.ds(i * LANES, LANES)]
            dst_all[j, pl.ds(i * LANES, LANES)] = jnp.where(s == d, trash, d)
        return 0
    lax.fori_loop(0, NBLK, _remap, 0)

    plsc.subcore_barrier()

    for b in range(KBUF):
        pltpu.async_copy(y_hbm.at[src_all.at[b]], rows[b], sems[b])

    def _group(g, _):
        j0 = g * KBUF
        for b in range(KBUF):
            j = j0 + b
            pltpu.make_async_copy(y_hbm.at[src_all.at[j]],
                                  rows[b], sems[b]).wait()
            pltpu.sync_copy(rows[b], agg_sh.at[dst_all.at[j]], add=True)
            pltpu.sync_copy(ones_buf, deg_sh.at[dst_all.at[j]], add=True)
            nj = j + KBUF

            @pl.when(nj < NBLK)
            def _():
                pltpu.async_copy(y_hbm.at[src_all.at[nj]], rows[b], sems[b])
        return 0

    lax.fori_loop(0, NBLK // KBUF, _group, 0)
    plsc.subcore_barrier()

    pltpu.sync_copy(agg_sh.at[pl.ds(row0, STRIPE)], stage_buf)
    pltpu.sync_copy(stage_buf, agg_out.at[cid, pl.ds(row0, STRIPE)])
    pltpu.sync_copy(deg_sh.at[pl.ds(row0, STRIPE)], deg_stage)
    pltpu.sync_copy(deg_stage, deg_out.at[cid, pl.ds(row0, STRIPE)])


def _post_body(y_ref, agg_ref, deg_ref, labels_ref, weights_ref,
               bf_ref, w1_ref, b1_ref, w2_ref, b2_ref,
               p_ref, loss_ref):
    agg = agg_ref[0] + agg_ref[1] + y_ref[...]
    deg = deg_ref[0, :, 0:1] + deg_ref[1, :, 0:1] + 1.0
    h = agg / deg + bf_ref[...]
    h = jnp.maximum(h, 0.0)
    h = jnp.dot(h, w1_ref[...], preferred_element_type=jnp.float32)
    h = jnp.maximum(h + b1_ref[...], 0.0)
    z = jnp.dot(h, w2_ref[...], preferred_element_type=jnp.float32)
    z = z + b2_ref[...]
    p = jax.nn.sigmoid(z)
    log_p = jnp.clip(jnp.log(p), -100.0)
    log_1mp = jnp.clip(jnp.log(1.0 - p), -100.0)
    lab = labels_ref[...]
    per = weights_ref[...] * -(lab * log_p + (1.0 - lab) * log_1mp)
    nrows = lax.broadcasted_iota(jnp.int32, (N_PAD, 1), 0)
    per = jnp.where(nrows < N_NODES, per, 0.0)
    p_ref[...] = p
    loss_ref[...] = (jnp.sum(per) / N_NODES).reshape(1, 1)


def kernel(in_, edge_index, labels, weights, W_feast, U, c, b_feast,
           W1, b1, W2, b2):
    del U, c

    x_pad = jnp.zeros((N_PAD, D_IN), jnp.float32).at[:N_NODES].set(in_)
    y_pad = pl.pallas_call(
        _matmul_body,
        out_shape=jax.ShapeDtypeStruct((N_PAD, D_HID), jnp.float32),
    )(x_pad, W_feast)

    src = edge_index[0].astype(jnp.int32)
    dst = edge_index[1].astype(jnp.int32)
    padn = E_PAD - N_EDGES
    zpad = jnp.zeros((padn,), jnp.int32)
    src_p = jnp.concatenate([src, zpad]).reshape(NW, NBLK, BE)
    dst_p = jnp.concatenate([dst, zpad]).reshape(NW, NBLK, BE)

    mesh = plsc.VectorSubcoreMesh(core_axis_name="c", subcore_axis_name="s",
                                  num_cores=NC, num_subcores=NS)
    agg_parts, deg_parts = pl.kernel(
        _edge_body,
        out_type=[
            jax.ShapeDtypeStruct((NC, N_PAD, D_HID), jnp.float32),
            jax.ShapeDtypeStruct((NC, N_PAD, DW), jnp.float32),
        ],
        mesh=mesh,
        compiler_params=pltpu.CompilerParams(use_tc_tiling_on_sc=False),
        scratch_types=[
            pltpu.VMEM((NBLK, BE), jnp.int32),
            pltpu.VMEM((NBLK, BE), jnp.int32),
            pltpu.VMEM((BE, DW), jnp.float32),
            pltpu.VMEM((STRIPE, D_HID), jnp.float32),
            pltpu.VMEM((STRIPE, DW), jnp.float32),
        ] + [pltpu.VMEM((BE, D_HID), jnp.float32)] * KBUF
          + [pltpu.SemaphoreType.DMA] * KBUF
          + [
            pltpu.VMEM_SHARED((N_PAD, D_HID), jnp.float32),
            pltpu.VMEM_SHARED((N_PAD, DW), jnp.float32),
        ],
    )(y_pad, src_p, dst_p,
      jnp.ones((BE, DW), jnp.float32),
      jnp.zeros((STRIPE, D_HID), jnp.float32),
      jnp.zeros((STRIPE, DW), jnp.float32))

    lab_pad = jnp.zeros((N_PAD, 1), jnp.float32).at[:N_NODES].set(labels)
    wt_pad = jnp.zeros((N_PAD, 1), jnp.float32).at[:N_NODES].set(weights)

    p_pad, loss_out = pl.pallas_call(
        _post_body,
        out_shape=[
            jax.ShapeDtypeStruct((N_PAD, 1), jnp.float32),
            jax.ShapeDtypeStruct((1, 1), jnp.float32),
        ],
    )(y_pad, agg_parts, deg_parts, lab_pad, wt_pad,
      b_feast.reshape(1, D_HID), W1, b1.reshape(1, 8), W2, b2.reshape(1, 1))

    return loss_out[0, 0], p_pad[:N_NODES]


# speedup vs baseline: 32.3730x; 1.4547x over previous
import jax
import jax.numpy as jnp
from jax import lax
from jax.experimental import pallas as pl
from jax.experimental.pallas import tpu as pltpu
from jax.experimental.pallas import tpu_sc as plsc

N_NODES = 10000
N_EDGES = 320000
D_IN = 128
D_HID = 16

NC = 2
NS = 16
NW = NC * NS
LANES = 16

BE = 128
DW = 8
KBUF = 8
NBLK = 80
N_PAD = 10240
STRIPE = N_PAD // NS
E_PAD = NW * NBLK * BE
EW = NBLK * BE
WIDE = N_PAD * D_HID // 128


def _matmul_body(x_ref, w_ref, y_ref):
    y_ref[0:N_NODES, :] = jnp.dot(x_ref[...], w_ref[...],
                                  preferred_element_type=jnp.float32)
    y_ref[N_NODES:N_PAD, :] = jnp.zeros((N_PAD - N_NODES, D_HID),
                                        jnp.float32)


def _edge_body(y_hbm, src_hbm, dst_hbm, ones_hbm, z16_hbm, z8_hbm,
               agg_out, deg_out,
               src_all, dst_all, ones_buf, stage_buf, deg_stage, ybuf,
               rows0, rows1, rows2, rows3, rows4, rows5, rows6, rows7,
               sem0, sem1, sem2, sem3, sem4, sem5, sem6, sem7,
               agg_sh, deg_sh):
    cid = lax.axis_index("c")
    sid = lax.axis_index("s")
    wid = cid * NS + sid
    trash = N_NODES + wid
    row0 = sid * STRIPE
    rows = (rows0, rows1, rows2, rows3, rows4, rows5, rows6, rows7)
    sems = (sem0, sem1, sem2, sem3, sem4, sem5, sem6, sem7)

    pltpu.sync_copy(src_hbm.at[wid], src_all)
    pltpu.sync_copy(dst_hbm.at[wid], dst_all)

    pltpu.sync_copy(ones_hbm, ones_buf)
    pltpu.sync_copy(z16_hbm, stage_buf)
    pltpu.sync_copy(z8_hbm, deg_stage)
    pltpu.sync_copy(stage_buf, agg_sh.at[pl.ds(row0, STRIPE)])
    pltpu.sync_copy(deg_stage, deg_sh.at[pl.ds(row0, STRIPE)])

    def _remap(j, _):
        for i in range(BE // LANES):
            s = src_all[j, pl.ds(i * LANES, LANES)]
            d = dst_all[j, pl.ds(i * LANES, LANES)]
            dst_all[j, pl.ds(i * LANES, LANES)] = jnp.where(s == d, trash, d)
        return 0
    lax.fori_loop(0, NBLK, _remap, 0)

    plsc.subcore_barrier()

    for b in range(KBUF):
        pltpu.async_copy(y_hbm.at[src_all.at[b]], rows[b], sems[b])

    def _group(g, _):
        j0 = g * KBUF
        for b in range(KBUF):
            j = j0 + b
            pltpu.make_async_copy(y_hbm.at[src_all.at[j]],
                                  rows[b], sems[b]).wait()
            pltpu.sync_copy(rows[b], agg_sh.at[dst_all.at[j]], add=True)
            pltpu.sync_copy(ones_buf, deg_sh.at[dst_all.at[j]], add=True)
            nj = j + KBUF

            @pl.when(nj < NBLK)
            def _():
                pltpu.async_copy(y_hbm.at[src_all.at[nj]], rows[b], sems[b])
        return 0

    lax.fori_loop(0, NBLK // KBUF, _group, 0)
    plsc.subcore_barrier()

    pltpu.sync_copy(agg_sh.at[pl.ds(row0, STRIPE)], stage_buf)

    @pl.when(cid == 0)
    def _():
        pltpu.sync_copy(y_hbm.at[pl.ds(row0, STRIPE)], ybuf)

        def _addy(i, _):
            stage_buf[i, :] = stage_buf[i, :] + ybuf[i, :]
            return 0
        lax.fori_loop(0, STRIPE, _addy, 0)

    pltpu.sync_copy(stage_buf, agg_out.at[cid, pl.ds(row0, STRIPE)])
    pltpu.sync_copy(deg_sh.at[pl.ds(row0, STRIPE)], deg_stage)
    pltpu.sync_copy(deg_stage,
                    deg_out.at[cid, pl.ds(row0, STRIPE), pl.ds(0, DW)])
    pltpu.sync_copy(deg_stage,
                    deg_out.at[cid, pl.ds(row0, STRIPE), pl.ds(DW, DW)])


def _post_body(agg_ref, deg_ref, lab_ref, wt_ref,
               bfw_ref, w1b_ref, b1w_ref, w2b_ref, b2w_ref,
               p_ref, loss_ref):
    agg = agg_ref[0] + agg_ref[1]
    deg = deg_ref[0] + deg_ref[1] + 1.0
    h = agg / deg + bfw_ref[...]
    h = jnp.maximum(h, 0.0)
    h2 = jnp.dot(h, w1b_ref[...], preferred_element_type=jnp.float32)
    h2 = jnp.maximum(h2 + b1w_ref[...], 0.0)
    z = jnp.dot(h2, w2b_ref[...], preferred_element_type=jnp.float32)
    z = z + b2w_ref[...]
    z_t = jnp.swapaxes(z, 0, 1)
    p = jax.nn.sigmoid(z_t)
    log_p = jnp.clip(jnp.log(p), -100.0)
    log_1mp = jnp.clip(jnp.log(1.0 - p), -100.0)
    lab = lab_ref[...]
    per = wt_ref[...] * -(lab * log_p + (1.0 - lab) * log_1mp)
    sub = lax.broadcasted_iota(jnp.int32, (8, WIDE), 0)
    lane = lax.broadcasted_iota(jnp.int32, (8, WIDE), 1)
    node = lane * 8 + sub
    per = jnp.where(node < N_NODES, per, 0.0)
    p_ref[...] = p
    loss_ref[...] = (jnp.sum(per) / N_NODES).reshape(1, 1)


def kernel(in_, edge_index, labels, weights, W_feast, U, c, b_feast,
           W1, b1, W2, b2):
    del U, c

    y_pad = pl.pallas_call(
        _matmul_body,
        out_shape=jax.ShapeDtypeStruct((N_PAD, D_HID), jnp.float32),
    )(in_, W_feast)

    src = edge_index[0].astype(jnp.int32)
    dst = edge_index[1].astype(jnp.int32)
    padn = E_PAD - N_EDGES
    zpad = jnp.zeros((padn,), jnp.int32)
    src_p = jnp.concatenate([src, zpad]).reshape(NW, NBLK, BE)
    dst_p = jnp.concatenate([dst, zpad]).reshape(NW, NBLK, BE)

    mesh = plsc.VectorSubcoreMesh(core_axis_name="c", subcore_axis_name="s",
                                  num_cores=NC, num_subcores=NS)
    agg_parts, deg_parts = pl.kernel(
        _edge_body,
        out_type=[
            jax.ShapeDtypeStruct((NC, N_PAD, D_HID), jnp.float32),
            jax.ShapeDtypeStruct((NC, N_PAD, 2 * DW), jnp.float32),
        ],
        mesh=mesh,
        compiler_params=pltpu.CompilerParams(use_tc_tiling_on_sc=False),
        scratch_types=[
            pltpu.VMEM((NBLK, BE), jnp.int32),
            pltpu.VMEM((NBLK, BE), jnp.int32),
            pltpu.VMEM((BE, DW), jnp.float32),
            pltpu.VMEM((STRIPE, D_HID), jnp.float32),
            pltpu.VMEM((STRIPE, DW), jnp.float32),
            pltpu.VMEM((STRIPE, D_HID), jnp.float32),
        ] + [pltpu.VMEM((BE, D_HID), jnp.float32)] * KBUF
          + [pltpu.SemaphoreType.DMA] * KBUF
          + [
            pltpu.VMEM_SHARED((N_PAD, D_HID), jnp.float32),
            pltpu.VMEM_SHARED((N_PAD, DW), jnp.float32),
        ],
    )(y_pad, src_p, dst_p,
      jnp.ones((BE, DW), jnp.float32),
      jnp.zeros((STRIPE, D_HID), jnp.float32),
      jnp.zeros((STRIPE, DW), jnp.float32))

    agg_w = agg_parts.reshape(NC, WIDE, 128)
    deg_w = deg_parts.reshape(NC, WIDE, 128)

    def _perm(v):
        vp = jnp.zeros((N_PAD,), jnp.float32).at[:N_NODES].set(v[:, 0])
        return vp.reshape(WIDE, 8).T

    lab_t = _perm(labels)
    wt_t = _perm(weights)

    eye8 = jnp.eye(8, dtype=jnp.float32)
    w1b = jnp.kron(eye8, W1)
    w2b = jnp.kron(eye8, W2)
    bfw = jnp.tile(b_feast, 8).reshape(1, 128)
    b1w = jnp.tile(b1, 8).reshape(1, 64)
    b2w = jnp.tile(b2, 8).reshape(1, 8)

    p_t, loss_out = pl.pallas_call(
        _post_body,
        out_shape=[
            jax.ShapeDtypeStruct((8, WIDE), jnp.float32),
            jax.ShapeDtypeStruct((1, 1), jnp.float32),
        ],
    )(agg_w, deg_w, lab_t, wt_t, bfw, w1b, b1w, w2b, b2w)

    p = p_t.T.reshape(N_PAD, 1)[:N_NODES]
    return loss_out[0, 0], p


# speedup vs baseline: 32.4198x; 1.0014x over previous
import jax
import jax.numpy as jnp
from jax import lax
from jax.experimental import pallas as pl
from jax.experimental.pallas import tpu as pltpu
from jax.experimental.pallas import tpu_sc as plsc

N_NODES = 10000
N_EDGES = 320000
D_IN = 128
D_HID = 16

NC = 2
NS = 16
NW = NC * NS
LANES = 16

BE = 128
DW = 8
KBUF = 8
NBLK = 80
N_PAD = 10240
STRIPE = N_PAD // NS
E_PAD = NW * NBLK * BE
EW = NBLK * BE
WIDE = N_PAD * D_HID // 128


def _matmul_body(x_ref, w_ref, y_ref):
    y_ref[0:N_NODES, :] = jnp.dot(x_ref[...], w_ref[...],
                                  preferred_element_type=jnp.float32)
    y_ref[N_NODES:N_PAD, :] = jnp.zeros((N_PAD - N_NODES, D_HID),
                                        jnp.float32)


def _edge_body(y_hbm, src_hbm, dst_hbm, ones_hbm, z16_hbm, z8_hbm,
               agg_out, deg_out,
               src_all, dst_all, ones_buf, stage_buf, deg_stage, ybuf,
               rows0, rows1, rows2, rows3, rows4, rows5, rows6, rows7,
               sem0, sem1, sem2, sem3, sem4, sem5, sem6, sem7,
               agg_sh, deg_sh):
    cid = lax.axis_index("c")
    sid = lax.axis_index("s")
    wid = cid * NS + sid
    trash = N_NODES + wid
    row0 = sid * STRIPE
    rows = (rows0, rows1, rows2, rows3, rows4, rows5, rows6, rows7)
    sems = (sem0, sem1, sem2, sem3, sem4, sem5, sem6, sem7)

    pltpu.sync_copy(src_hbm.at[wid], src_all)
    pltpu.sync_copy(dst_hbm.at[wid], dst_all)

    pltpu.sync_copy(ones_hbm, ones_buf)
    pltpu.sync_copy(z16_hbm, stage_buf)
    pltpu.sync_copy(z8_hbm, deg_stage)
    pltpu.sync_copy(stage_buf, agg_sh.at[pl.ds(row0, STRIPE)])
    pltpu.sync_copy(deg_stage, deg_sh.at[pl.ds(row0, STRIPE)])

    def _remap(j, _):
        for i in range(BE // LANES):
            s = src_all[j, pl.ds(i * LANES, LANES)]
            d = dst_all[j, pl.ds(i * LANES, LANES)]
            dst_all[j, pl.ds(i * LANES, LANES)] = jnp.where(s == d, trash, d)
        return 0
    lax.fori_loop(0, NBLK, _remap, 0)

    plsc.subcore_barrier()

    for b in range(KBUF):
        pltpu.async_copy(y_hbm.at[src_all.at[b]], rows[b], sems[b])

    def _group(g, _):
        j0 = g * KBUF
        for b in range(KBUF):
            j = j0 + b
            pltpu.make_async_copy(y_hbm.at[src_all.at[j]],
                                  rows[b], sems[b]).wait()
            pltpu.sync_copy(rows[b], agg_sh.at[dst_all.at[j]], add=True)
            pltpu.sync_copy(ones_buf, deg_sh.at[dst_all.at[j]], add=True)
            nj = j + KBUF

            @pl.when(nj < NBLK)
            def _():
                pltpu.async_copy(y_hbm.at[src_all.at[nj]], rows[b], sems[b])
        return 0

    lax.fori_loop(0, NBLK // KBUF, _group, 0)
    plsc.subcore_barrier()

    pltpu.sync_copy(agg_sh.at[pl.ds(row0, STRIPE)], stage_buf)

    @pl.when(cid == 0)
    def _():
        pltpu.sync_copy(y_hbm.at[pl.ds(row0, STRIPE)], ybuf)

        def _addy(i, _):
            stage_buf[i, :] = stage_buf[i, :] + ybuf[i, :]
            return 0
        lax.fori_loop(0, STRIPE, _addy, 0)

    pltpu.sync_copy(stage_buf, agg_out.at[cid, pl.ds(row0, STRIPE)])
    pltpu.sync_copy(deg_sh.at[pl.ds(row0, STRIPE)], deg_stage)
    pltpu.sync_copy(deg_stage,
                    deg_out.at[cid, pl.ds(row0, STRIPE), pl.ds(0, DW)])
    pltpu.sync_copy(deg_stage,
                    deg_out.at[cid, pl.ds(row0, STRIPE), pl.ds(DW, DW)])


def _post_body(agg_ref, deg_ref, lab_ref, wt_ref,
               bfw_ref, w1b_ref, b1w_ref, w2b_ref, b2w_ref,
               p_ref, loss_ref):
    agg = agg_ref[0] + agg_ref[1]
    deg = deg_ref[0] + deg_ref[1] + 1.0
    h = agg / deg + bfw_ref[...]
    h = jnp.maximum(h, 0.0)
    h2 = jnp.dot(h, w1b_ref[...], preferred_element_type=jnp.float32)
    h2 = jnp.maximum(h2 + b1w_ref[...], 0.0)
    z = jnp.dot(h2, w2b_ref[...], preferred_element_type=jnp.float32)
    z = z + b2w_ref[...]
    z_t = jnp.swapaxes(z, 0, 1)
    p = jax.nn.sigmoid(z_t)
    log_p = jnp.clip(jnp.log(p), -100.0)
    log_1mp = jnp.clip(jnp.log(1.0 - p), -100.0)
    lab = lab_ref[...]
    per = wt_ref[...] * -(lab * log_p + (1.0 - lab) * log_1mp)
    sub = lax.broadcasted_iota(jnp.int32, (8, WIDE), 0)
    lane = lax.broadcasted_iota(jnp.int32, (8, WIDE), 1)
    node = lane * 8 + sub
    per = jnp.where(node < N_NODES, per, 0.0)
    p_ref[...] = p
    loss_ref[...] = (jnp.sum(per) / N_NODES).reshape(1, 1)


def kernel(in_, edge_index, labels, weights, W_feast, U, c, b_feast,
           W1, b1, W2, b2):
    del U, c

    y_pad = pl.pallas_call(
        _matmul_body,
        out_shape=jax.ShapeDtypeStruct((N_PAD, D_HID), jnp.float32),
    )(in_, W_feast)

    src = edge_index[0].astype(jnp.int32)
    dst = edge_index[1].astype(jnp.int32)
    padn = E_PAD - N_EDGES
    zpad = jnp.zeros((padn,), jnp.int32)
    dpad = N_NODES + (jnp.arange(padn, dtype=jnp.int32) % (N_PAD - N_NODES))
    src_p = jnp.concatenate([src, zpad]).reshape(NW, NBLK, BE)
    dst_p = jnp.concatenate([dst, dpad]).reshape(NW, NBLK, BE)

    mesh = plsc.VectorSubcoreMesh(core_axis_name="c", subcore_axis_name="s",
                                  num_cores=NC, num_subcores=NS)
    agg_parts, deg_parts = pl.kernel(
        _edge_body,
        out_type=[
            jax.ShapeDtypeStruct((NC, N_PAD, D_HID), jnp.float32),
            jax.ShapeDtypeStruct((NC, N_PAD, 2 * DW), jnp.float32),
        ],
        mesh=mesh,
        compiler_params=pltpu.CompilerParams(use_tc_tiling_on_sc=False),
        scratch_types=[
            pltpu.VMEM((NBLK, BE), jnp.int32),
            pltpu.VMEM((NBLK, BE), jnp.int32),
            pltpu.VMEM((BE, DW), jnp.float32),
            pltpu.VMEM((STRIPE, D_HID), jnp.float32),
            pltpu.VMEM((STRIPE, DW), jnp.float32),
            pltpu.VMEM((STRIPE, D_HID), jnp.float32),
        ] + [pltpu.VMEM((BE, D_HID), jnp.float32)] * KBUF
          + [pltpu.SemaphoreType.DMA] * KBUF
          + [
            pltpu.VMEM_SHARED((N_PAD, D_HID), jnp.float32),
            pltpu.VMEM_SHARED((N_PAD, DW), jnp.float32),
        ],
    )(y_pad, src_p, dst_p,
      jnp.ones((BE, DW), jnp.float32),
      jnp.zeros((STRIPE, D_HID), jnp.float32),
      jnp.zeros((STRIPE, DW), jnp.float32))

    agg_w = agg_parts.reshape(NC, WIDE, 128)
    deg_w = deg_parts.reshape(NC, WIDE, 128)

    def _perm(v):
        vp = jnp.zeros((N_PAD,), jnp.float32).at[:N_NODES].set(v[:, 0])
        return vp.reshape(WIDE, 8).T

    lab_t = _perm(labels)
    wt_t = _perm(weights)

    eye8 = jnp.eye(8, dtype=jnp.float32)
    w1b = jnp.kron(eye8, W1)
    w2b = jnp.kron(eye8, W2)
    bfw = jnp.tile(b_feast, 8).reshape(1, 128)
    b1w = jnp.tile(b1, 8).reshape(1, 64)
    b2w = jnp.tile(b2, 8).reshape(1, 8)

    p_t, loss_out = pl.pallas_call(
        _post_body,
        out_shape=[
            jax.ShapeDtypeStruct((8, WIDE), jnp.float32),
            jax.ShapeDtypeStruct((1, 1), jnp.float32),
        ],
    )(agg_w, deg_w, lab_t, wt_t, bfw, w1b, b1w, w2b, b2w)

    p = p_t.T.reshape(N_PAD, 1)[:N_NODES]
    return loss_out[0, 0], p


# speedup vs baseline: 46.6645x; 1.4394x over previous
import jax
import jax.numpy as jnp
from jax import lax
from jax.experimental import pallas as pl
from jax.experimental.pallas import tpu as pltpu
from jax.experimental.pallas import tpu_sc as plsc

N_NODES = 10000
N_EDGES = 320000
D_IN = 128
D_HID = 16

NC = 2
NS = 16
NW = NC * NS
LANES = 16

BE = 128
DW = 8
KBUF = 8
NBLK = 79
NDUP = NW * NBLK * BE // BE - N_EDGES // BE
N_PAD = 10240
STRIPE = N_PAD // NS
EW = NBLK * BE
WIDE = N_PAD * D_HID // 128


def _matmul_body(x_ref, w_ref, y_ref):
    y_ref[0:N_NODES, :] = jnp.dot(x_ref[...], w_ref[...],
                                  preferred_element_type=jnp.float32)
    y_ref[N_NODES:N_PAD, :] = jnp.zeros((N_PAD - N_NODES, D_HID),
                                        jnp.float32)


def _edge_body(y_hbm, ei_hbm, ones_hbm, z16_hbm, z8_hbm,
               agg_out, deg_out,
               src_all, dst_all, dstp, ones_buf, stage_buf, deg_stage, ybuf,
               rows0, rows1, rows2, rows3, rows4, rows5, rows6, rows7,
               sem0, sem1, sem2, sem3, sem4, sem5, sem6, sem7,
               agg_sh, deg_sh):
    cid = lax.axis_index("c")
    sid = lax.axis_index("s")
    wid = cid * NS + sid
    trash = N_NODES + wid
    row0 = sid * STRIPE
    rows = (rows0, rows1, rows2, rows3, rows4, rows5, rows6, rows7)
    sems = (sem0, sem1, sem2, sem3, sem4, sem5, sem6, sem7)

    last = wid == NW - 1
    base = jnp.where(last, N_EDGES - EW, wid * EW)
    pltpu.sync_copy(ei_hbm.at[0, pl.ds(base, EW)], src_all)
    pltpu.sync_copy(ei_hbm.at[1, pl.ds(base, EW)], dst_all)

    pltpu.sync_copy(ones_hbm, ones_buf)
    pltpu.sync_copy(z16_hbm, stage_buf)
    pltpu.sync_copy(z8_hbm, deg_stage)
    pltpu.sync_copy(stage_buf, agg_sh.at[pl.ds(row0, STRIPE)])
    pltpu.sync_copy(deg_stage, deg_sh.at[pl.ds(row0, STRIPE)])

    def _remap(j, _):
        dup = jnp.where(last, 1, 0) * jnp.where(j < NDUP, 1, 0)
        for i in range(BE // LANES):
            off = j * BE + i * LANES
            s = src_all[pl.ds(off, LANES)]
            d = dst_all[pl.ds(off, LANES)]
            d_eff = d * (1 - dup) + trash * dup
            dstp[j, pl.ds(i * LANES, LANES)] = jnp.where(s == d, trash, d_eff)
        return 0
    lax.fori_loop(0, NBLK, _remap, 0)

    plsc.subcore_barrier()

    for b in range(KBUF):
        pltpu.async_copy(y_hbm.at[src_all.at[pl.ds(b * BE, BE)]],
                         rows[b], sems[b])

    def _group(g, _):
        j0 = g * KBUF
        for b in range(KBUF):
            j = j0 + b

            @pl.when(j < NBLK)
            def _():
                pltpu.make_async_copy(
                    y_hbm.at[src_all.at[pl.ds(j * BE, BE)]],
                    rows[b], sems[b]).wait()
                pltpu.sync_copy(rows[b], agg_sh.at[dstp.at[j]], add=True)
                pltpu.sync_copy(ones_buf, deg_sh.at[dstp.at[j]], add=True)
                nj = j + KBUF

                @pl.when(nj < NBLK)
                def _():
                    pltpu.async_copy(
                        y_hbm.at[src_all.at[pl.ds(nj * BE, BE)]],
                        rows[b], sems[b])
        return 0

    lax.fori_loop(0, (NBLK + KBUF - 1) // KBUF, _group, 0)
    plsc.subcore_barrier()

    pltpu.sync_copy(agg_sh.at[pl.ds(row0, STRIPE)], stage_buf)

    @pl.when(cid == 0)
    def _():
        pltpu.sync_copy(y_hbm.at[pl.ds(row0, STRIPE)], ybuf)

        def _addy(i, _):
            stage_buf[i, :] = stage_buf[i, :] + ybuf[i, :]
            return 0
        lax.fori_loop(0, STRIPE, _addy, 0)

    pltpu.sync_copy(stage_buf, agg_out.at[cid, pl.ds(row0, STRIPE)])
    pltpu.sync_copy(deg_sh.at[pl.ds(row0, STRIPE)], deg_stage)
    pltpu.sync_copy(deg_stage,
                    deg_out.at[cid, pl.ds(row0, STRIPE), pl.ds(0, DW)])
    pltpu.sync_copy(deg_stage,
                    deg_out.at[cid, pl.ds(row0, STRIPE), pl.ds(DW, DW)])


def _post_body(agg_ref, deg_ref, lab_ref, wt_ref,
               bfw_ref, w1b_ref, b1w_ref, w2b_ref, b2w_ref,
               p_ref, loss_ref):
    agg = agg_ref[0] + agg_ref[1]
    deg = deg_ref[0] + deg_ref[1] + 1.0
    h = agg / deg + bfw_ref[...]
    h = jnp.maximum(h, 0.0)
    h2 = jnp.dot(h, w1b_ref[...], preferred_element_type=jnp.float32)
    h2 = jnp.maximum(h2 + b1w_ref[...], 0.0)
    z = jnp.dot(h2, w2b_ref[...], preferred_element_type=jnp.float32)
    z = z + b2w_ref[...]
    z_t = jnp.swapaxes(z, 0, 1)
    p = jax.nn.sigmoid(z_t)
    log_p = jnp.clip(jnp.log(p), -100.0)
    log_1mp = jnp.clip(jnp.log(1.0 - p), -100.0)
    lab = lab_ref[...]
    per = wt_ref[...] * -(lab * log_p + (1.0 - lab) * log_1mp)
    sub = lax.broadcasted_iota(jnp.int32, (8, WIDE), 0)
    lane = lax.broadcasted_iota(jnp.int32, (8, WIDE), 1)
    node = lane * 8 + sub
    per = jnp.where(node < N_NODES, per, 0.0)
    p_ref[...] = p
    loss_ref[...] = (jnp.sum(per) / N_NODES).reshape(1, 1)


def kernel(in_, edge_index, labels, weights, W_feast, U, c, b_feast,
           W1, b1, W2, b2):
    del U, c

    y_pad = pl.pallas_call(
        _matmul_body,
        out_shape=jax.ShapeDtypeStruct((N_PAD, D_HID), jnp.float32),
    )(in_, W_feast)

    ei32 = edge_index.astype(jnp.int32)

    mesh = plsc.VectorSubcoreMesh(core_axis_name="c", subcore_axis_name="s",
                                  num_cores=NC, num_subcores=NS)
    agg_parts, deg_parts = pl.kernel(
        _edge_body,
        out_type=[
            jax.ShapeDtypeStruct((NC, N_PAD, D_HID), jnp.float32),
            jax.ShapeDtypeStruct((NC, N_PAD, 2 * DW), jnp.float32),
        ],
        mesh=mesh,
        compiler_params=pltpu.CompilerParams(use_tc_tiling_on_sc=False),
        scratch_types=[
            pltpu.VMEM((EW,), jnp.int32),
            pltpu.VMEM((EW,), jnp.int32),
            pltpu.VMEM((NBLK, BE), jnp.int32),
            pltpu.VMEM((BE, DW), jnp.float32),
            pltpu.VMEM((STRIPE, D_HID), jnp.float32),
            pltpu.VMEM((STRIPE, DW), jnp.float32),
            pltpu.VMEM((STRIPE, D_HID), jnp.float32),
        ] + [pltpu.VMEM((BE, D_HID), jnp.float32)] * KBUF
          + [pltpu.SemaphoreType.DMA] * KBUF
          + [
            pltpu.VMEM_SHARED((N_PAD, D_HID), jnp.float32),
            pltpu.VMEM_SHARED((N_PAD, DW), jnp.float32),
        ],
    )(y_pad, ei32,
      jnp.ones((BE, DW), jnp.float32),
      jnp.zeros((STRIPE, D_HID), jnp.float32),
      jnp.zeros((STRIPE, DW), jnp.float32))

    agg_w = agg_parts.reshape(NC, WIDE, 128)
    deg_w = deg_parts.reshape(NC, WIDE, 128)

    def _perm(v):
        vp = jnp.zeros((N_PAD,), jnp.float32).at[:N_NODES].set(v[:, 0])
        return vp.reshape(WIDE, 8).T

    lab_t = _perm(labels)
    wt_t = _perm(weights)

    eye8 = jnp.eye(8, dtype=jnp.float32)
    w1b = jnp.kron(eye8, W1)
    w2b = jnp.kron(eye8, W2)
    bfw = jnp.tile(b_feast, 8).reshape(1, 128)
    b1w = jnp.tile(b1, 8).reshape(1, 64)
    b2w = jnp.tile(b2, 8).reshape(1, 8)

    p_t, loss_out = pl.pallas_call(
        _post_body,
        out_shape=[
            jax.ShapeDtypeStruct((8, WIDE), jnp.float32),
            jax.ShapeDtypeStruct((1, 1), jnp.float32),
        ],
    )(agg_w, deg_w, lab_t, wt_t, bfw, w1b, b1w, w2b, b2w)

    p = p_t.T.reshape(N_PAD, 1)[:N_NODES]
    return loss_out[0, 0], p


# speedup vs baseline: 47.2448x; 1.0124x over previous
import jax
import jax.numpy as jnp
import numpy as np
from jax import lax
from jax.experimental import pallas as pl
from jax.experimental.pallas import tpu as pltpu
from jax.experimental.pallas import tpu_sc as plsc

N_NODES = 10000
N_EDGES = 320000
D_IN = 128
D_HID = 16

NC = 2
NS = 16
NW = NC * NS
LANES = 16

BE = 128
DW = 8
KBUF = 8
NBLK = 79
NDUP = NW * NBLK * BE // BE - N_EDGES // BE
N_PAD = 10240
STRIPE = N_PAD // NS
EW = NBLK * BE
WIDE = N_PAD * D_HID // 128


def _matmul_body(x_ref, w_ref, y_ref):
    y_ref[0:N_NODES, :] = jnp.dot(x_ref[...], w_ref[...],
                                  preferred_element_type=jnp.float32)
    y_ref[N_NODES:N_PAD, :] = jnp.zeros((N_PAD - N_NODES, D_HID),
                                        jnp.float32)


def _edge_body(y_hbm, ei_hbm, ones_hbm, z16_hbm, z8_hbm,
               agg_out, deg_out,
               src_all, dst_all, dstp, ones_buf, stage_buf, deg_stage, ybuf,
               rows0, rows1, rows2, rows3, rows4, rows5, rows6, rows7,
               sem0, sem1, sem2, sem3, sem4, sem5, sem6, sem7, dsem,
               agg_sh, deg_sh):
    cid = lax.axis_index("c")
    sid = lax.axis_index("s")
    wid = cid * NS + sid
    trash = N_NODES + wid
    row0 = sid * STRIPE
    rows = (rows0, rows1, rows2, rows3, rows4, rows5, rows6, rows7)
    sems = (sem0, sem1, sem2, sem3, sem4, sem5, sem6, sem7)

    last = wid == NW - 1
    base = jnp.where(last, N_EDGES - EW, wid * EW)
    pltpu.sync_copy(ei_hbm.at[0, pl.ds(base, EW)], src_all)
    pltpu.sync_copy(ei_hbm.at[1, pl.ds(base, EW)], dst_all)

    pltpu.sync_copy(ones_hbm, ones_buf)
    pltpu.sync_copy(z16_hbm, stage_buf)
    pltpu.sync_copy(z8_hbm, deg_stage)
    pltpu.sync_copy(stage_buf, agg_sh.at[pl.ds(row0, STRIPE)])
    pltpu.sync_copy(deg_stage, deg_sh.at[pl.ds(row0, STRIPE)])

    def _remap(j, _):
        dup = jnp.where(last, 1, 0) * jnp.where(j < NDUP, 1, 0)
        for i in range(BE // LANES):
            off = j * BE + i * LANES
            s = src_all[pl.ds(off, LANES)]
            d = dst_all[pl.ds(off, LANES)]
            d_eff = d * (1 - dup) + trash * dup
            dstp[j, pl.ds(i * LANES, LANES)] = jnp.where(s == d, trash, d_eff)
        return 0
    lax.fori_loop(0, NBLK, _remap, 0)

    plsc.subcore_barrier()

    for b in range(KBUF):
        pltpu.async_copy(y_hbm.at[src_all.at[pl.ds(b * BE, BE)]],
                         rows[b], sems[b])

    def _group(g, _):
        j0 = g * KBUF
        for b in range(KBUF):
            j = j0 + b

            @pl.when(j < NBLK)
            def _():
                pltpu.make_async_copy(
                    y_hbm.at[src_all.at[pl.ds(j * BE, BE)]],
                    rows[b], sems[b]).wait()
                pltpu.sync_copy(rows[b], agg_sh.at[dstp.at[j]], add=True)
                pltpu.async_copy(ones_buf, deg_sh.at[dstp.at[j]], dsem,
                                 add=True)
                nj = j + KBUF

                @pl.when(nj < NBLK)
                def _():
                    pltpu.async_copy(
                        y_hbm.at[src_all.at[pl.ds(nj * BE, BE)]],
                        rows[b], sems[b])
        return 0

    lax.fori_loop(0, (NBLK + KBUF - 1) // KBUF, _group, 0)

    def _drain(j, _):
        pltpu.make_async_copy(ones_buf, deg_sh.at[dstp.at[0]], dsem).wait()
        return 0
    lax.fori_loop(0, NBLK, _drain, 0)

    plsc.subcore_barrier()

    pltpu.sync_copy(agg_sh.at[pl.ds(row0, STRIPE)], stage_buf)

    @pl.when(cid == 0)
    def _():
        pltpu.sync_copy(y_hbm.at[pl.ds(row0, STRIPE)], ybuf)

        def _addy(i, _):
            stage_buf[i, :] = stage_buf[i, :] + ybuf[i, :]
            return 0
        lax.fori_loop(0, STRIPE, _addy, 0)

    pltpu.sync_copy(stage_buf, agg_out.at[cid, pl.ds(row0, STRIPE)])
    pltpu.sync_copy(deg_sh.at[pl.ds(row0, STRIPE)], deg_stage)
    pltpu.sync_copy(deg_stage,
                    deg_out.at[cid, pl.ds(row0, STRIPE), pl.ds(0, DW)])
    pltpu.sync_copy(deg_stage,
                    deg_out.at[cid, pl.ds(row0, STRIPE), pl.ds(DW, DW)])


def _post_body(agg_ref, deg_ref, lab_ref, wt_ref,
               bfw_ref, w1b_ref, b1w_ref, w2b_ref, b2w_ref,
               p_ref, loss_ref):
    agg = agg_ref[0] + agg_ref[1]
    deg = deg_ref[0] + deg_ref[1] + 1.0
    h = agg / deg + bfw_ref[...]
    h = jnp.maximum(h, 0.0)
    h2 = jnp.dot(h, w1b_ref[...], preferred_element_type=jnp.float32)
    h2 = jnp.maximum(h2 + b1w_ref[...], 0.0)
    z = jnp.dot(h2, w2b_ref[...], preferred_element_type=jnp.float32)
    z = z + b2w_ref[...]
    z_t = jnp.swapaxes(z, 0, 1)
    p = jax.nn.sigmoid(z_t)
    log_p = jnp.clip(jnp.log(p), -100.0)
    log_1mp = jnp.clip(jnp.log(1.0 - p), -100.0)
    lab = lab_ref[...]
    per = wt_ref[...] * -(lab * log_p + (1.0 - lab) * log_1mp)
    sub = lax.broadcasted_iota(jnp.int32, (8, WIDE), 0)
    lane = lax.broadcasted_iota(jnp.int32, (8, WIDE), 1)
    node = lane * 8 + sub
    per = jnp.where(node < N_NODES, per, 0.0)
    p_ref[...] = p
    loss_ref[...] = (jnp.sum(per) / N_NODES).reshape(1, 1)


def kernel(in_, edge_index, labels, weights, W_feast, U, c, b_feast,
           W1, b1, W2, b2):
    del U, c

    y_pad = pl.pallas_call(
        _matmul_body,
        out_shape=jax.ShapeDtypeStruct((N_PAD, D_HID), jnp.float32),
    )(in_, W_feast)

    ei32 = edge_index.astype(jnp.int32)

    mesh = plsc.VectorSubcoreMesh(core_axis_name="c", subcore_axis_name="s",
                                  num_cores=NC, num_subcores=NS)
    agg_parts, deg_parts = pl.kernel(
        _edge_body,
        out_type=[
            jax.ShapeDtypeStruct((NC, N_PAD, D_HID), jnp.float32),
            jax.ShapeDtypeStruct((NC, N_PAD, 2 * DW), jnp.float32),
        ],
        mesh=mesh,
        compiler_params=pltpu.CompilerParams(use_tc_tiling_on_sc=False),
        scratch_types=[
            pltpu.VMEM((EW,), jnp.int32),
            pltpu.VMEM((EW,), jnp.int32),
            pltpu.VMEM((NBLK, BE), jnp.int32),
            pltpu.VMEM((BE, DW), jnp.float32),
            pltpu.VMEM((STRIPE, D_HID), jnp.float32),
            pltpu.VMEM((STRIPE, DW), jnp.float32),
            pltpu.VMEM((STRIPE, D_HID), jnp.float32),
        ] + [pltpu.VMEM((BE, D_HID), jnp.float32)] * KBUF
          + [pltpu.SemaphoreType.DMA] * (KBUF + 1)
          + [
            pltpu.VMEM_SHARED((N_PAD, D_HID), jnp.float32),
            pltpu.VMEM_SHARED((N_PAD, DW), jnp.float32),
        ],
    )(y_pad, ei32,
      np.ones((BE, DW), np.float32),
      np.zeros((STRIPE, D_HID), np.float32),
      np.zeros((STRIPE, DW), np.float32))

    agg_w = agg_parts.reshape(NC, WIDE, 128)
    deg_w = deg_parts.reshape(NC, WIDE, 128)

    def _perm(v):
        vp = jnp.zeros((N_PAD,), jnp.float32).at[:N_NODES].set(v[:, 0])
        return vp.reshape(WIDE, 8).T

    lab_t = _perm(labels)
    wt_t = _perm(weights)

    eye8 = jnp.eye(8, dtype=jnp.float32)
    w1b = jnp.kron(eye8, W1)
    w2b = jnp.kron(eye8, W2)
    bfw = jnp.tile(b_feast, 8).reshape(1, 128)
    b1w = jnp.tile(b1, 8).reshape(1, 64)
    b2w = jnp.tile(b2, 8).reshape(1, 8)

    p_t, loss_out = pl.pallas_call(
        _post_body,
        out_shape=[
            jax.ShapeDtypeStruct((8, WIDE), jnp.float32),
            jax.ShapeDtypeStruct((1, 1), jnp.float32),
        ],
    )(agg_w, deg_w, lab_t, wt_t, bfw, w1b, b1w, w2b, b2w)

    p = p_t.T.reshape(N_PAD, 1)[:N_NODES]
    return loss_out[0, 0], p
